# baseline (speedup 1.0000x reference)
import functools
import math

import jax
import jax.numpy as jnp
from jax import lax
from jax.experimental import pallas as pl
from jax.experimental.pallas import tpu as pltpu
from jax.experimental.pallas import tpu_sc as plsc

VOCAB = 100000
D = 512
BATCH = 4096
SEQ = 200
N = BATCH * SEQ
NC, NS = 2, 16
NW = NC * NS
ROWS_W = N // NW
CHUNK = 32
NCHUNK = ROWS_W // CHUNK
NBUF = 5
SCALE = math.sqrt(float(D))

_mesh = plsc.VectorSubcoreMesh(
    core_axis_name="c", subcore_axis_name="s", num_cores=NC, num_subcores=NS
)


@functools.partial(
    pl.kernel,
    out_type=jax.ShapeDtypeStruct((N, D), jnp.float32),
    mesh=_mesh,
    scratch_types=[
        pltpu.VMEM((ROWS_W,), jnp.int32),
    ]
    + [pltpu.VMEM((CHUNK, D), jnp.float32) for _ in range(NBUF)]
    + [pltpu.SemaphoreType.DMA for _ in range(2 * NBUF)],
)
def _sc_gather(table_hbm, idx_hbm, out_hbm, idx_v, *bufs_and_sems):
    rows = bufs_and_sems[:NBUF]
    gsem = bufs_and_sems[NBUF : 2 * NBUF]
    ssem = bufs_and_sems[2 * NBUF :]

    wid = lax.axis_index("s") * NC + lax.axis_index("c")
    base = wid * ROWS_W

    pltpu.sync_copy(idx_hbm.at[pl.ds(base, ROWS_W)], idx_v)

    def idx_slice(c):
        return idx_v.at[pl.ds(c * CHUNK, CHUNK)]

    def out_slice(c):
        return out_hbm.at[pl.ds(base + c * CHUNK, CHUNK)]

    for b in range(NBUF):
        pltpu.async_copy(table_hbm.at[idx_slice(b)], rows[b], gsem[b])

    @pl.loop(0, NCHUNK, step=NBUF)
    def _super(c):
        for b in range(NBUF):
            k = c + b
            pltpu.make_async_copy(table_hbm.at[idx_slice(k)], rows[b], gsem[b]).wait()

            @pl.loop(0, CHUNK)
            def _scale_row(r, _b=b):
                row = rows[_b]
                for j in range(D // 16):
                    sl = pl.ds(j * 16, 16)
                    row[r, sl] = row[r, sl] * jnp.float32(SCALE)

            pltpu.async_copy(rows[b], out_slice(k), ssem[b])
        for b in range(NBUF):
            k = c + b + NBUF

            @pl.when(k < NCHUNK)
            def _():
                pltpu.make_async_copy(rows[b], out_slice(k - NBUF), ssem[b]).wait()
                pltpu.async_copy(table_hbm.at[idx_slice(k)], rows[b], gsem[b])

    for b in range(NBUF):
        pltpu.make_async_copy(rows[b], out_slice(NCHUNK - NBUF + b), ssem[b]).wait()


def kernel(x, table):
    out = _sc_gather(table, x.reshape(N))
    return out.reshape(BATCH, SEQ, D)
